# baseline (speedup 1.0000x reference)
import jax
import jax.numpy as jnp
from jax import lax
from jax.experimental import pallas as pl
from jax.experimental.pallas import tpu as pltpu
from jax.experimental.pallas import tpu_sc as plsc

BATCH = 4096
HIST = 50
DIM = 64
VOCAB = 100000
NUM_CORES = 2
NUM_SUBCORES = 16
NW = NUM_CORES * NUM_SUBCORES
FEAT_W = DIM // NW
NCHUNK = BATCH // 16
UNROLL = 8


def _emb_body(idx_hbm, table_hbm, out_hbm, wrow,
              ib0, ib1, ob0, ob1, is0, is1, os0, os1):
    cid = lax.axis_index("c")
    sid = lax.axis_index("s")
    w = sid * NUM_CORES + cid

    ibufs = (ib0, ib1)
    obufs = (ob0, ob1)
    isems = (is0, is1)
    osems = (os0, os1)
    ih = [None, None]
    oh = [None, None]

    for dd in range(FEAT_W):
        d = FEAT_W * w + dd
        pltpu.sync_copy(table_hbm.at[d], wrow)
        ih[0] = pltpu.async_copy(idx_hbm.at[0], ibufs[0], isems[0])
        for h in range(HIST):
            par = h % 2
            if h + 1 < HIST:
                ih[1 - par] = pltpu.async_copy(
                    idx_hbm.at[h + 1], ibufs[1 - par], isems[1 - par])
            ih[par].wait()
            if oh[par] is not None:
                oh[par].wait()
            idx_v = ibufs[par]
            out_v = obufs[par]

            def chunk(cc, _, idx_v=idx_v, out_v=out_v):
                for u in range(UNROLL):
                    off = (cc * UNROLL + u) * 16
                    i16 = idx_v[pl.ds(off, 16)]
                    out_v[pl.ds(off, 16)] = plsc.load_gather(wrow, [i16])
                return ()

            lax.fori_loop(0, NCHUNK // UNROLL, chunk, ())
            oh[par] = pltpu.async_copy(out_v, out_hbm.at[h, d], osems[par])
    oh[0].wait()
    oh[1].wait()


def kernel(input, weight):
    mesh = plsc.VectorSubcoreMesh(core_axis_name="c", subcore_axis_name="s")
    out_t = pl.kernel(
        _emb_body,
        out_type=jax.ShapeDtypeStruct((HIST, DIM, BATCH), jnp.float32),
        mesh=mesh,
        scratch_types=[
            pltpu.VMEM((VOCAB,), jnp.float32),
            pltpu.VMEM((BATCH,), jnp.int32),
            pltpu.VMEM((BATCH,), jnp.int32),
            pltpu.VMEM((BATCH,), jnp.float32),
            pltpu.VMEM((BATCH,), jnp.float32),
            pltpu.SemaphoreType.DMA,
            pltpu.SemaphoreType.DMA,
            pltpu.SemaphoreType.DMA,
            pltpu.SemaphoreType.DMA,
        ],
        compiler_params=pltpu.CompilerParams(use_tc_tiling_on_sc=False,
                                             needs_layout_passes=False),
    )(input.T, weight.T)
    return out_t.transpose(2, 0, 1)


# speedup vs baseline: 1.1863x; 1.1863x over previous
import jax
import jax.numpy as jnp
from jax import lax
from jax.experimental import pallas as pl
from jax.experimental.pallas import tpu as pltpu
from jax.experimental.pallas import tpu_sc as plsc

BATCH = 4096
HIST = 50
DIM = 64
VOCAB = 100000
NUM_CORES = 2
NUM_SUBCORES = 16
NW = NUM_CORES * NUM_SUBCORES
FEAT_W = DIM // NW
NCHUNK = BATCH // 16
UNROLL = 8


def _emb_body(idx_hbm, table_hbm, out_hbm, wrow,
              ib0, ib1, ob0, ob1, is0, is1, os0, os1):
    cid = lax.axis_index("c")
    sid = lax.axis_index("s")
    w = sid * NUM_CORES + cid

    ibufs = (ib0, ib1)
    obufs = (ob0, ob1)
    isems = (is0, is1)
    osems = (os0, os1)
    ih = [None, None]
    oh = [None, None]

    for dd in range(FEAT_W):
        d = FEAT_W * w + dd
        pltpu.sync_copy(table_hbm.at[d], wrow)
        ih[0] = pltpu.async_copy(idx_hbm.at[0], ibufs[0], isems[0])
        for h in range(HIST):
            par = h % 2
            if h + 1 < HIST:
                ih[1 - par] = pltpu.async_copy(
                    idx_hbm.at[h + 1], ibufs[1 - par], isems[1 - par])
            ih[par].wait()
            if oh[par] is not None:
                oh[par].wait()
            idx_v = ibufs[par]
            out_v = obufs[par]

            @plsc.parallel_loop(0, BATCH, step=16, unroll=UNROLL)
            def _gather(off, idx_v=idx_v, out_v=out_v):
                i16 = idx_v[pl.ds(off, 16)]
                out_v[pl.ds(off, 16)] = plsc.load_gather(wrow, [i16])
            oh[par] = pltpu.async_copy(out_v, out_hbm.at[h, d], osems[par])
    oh[0].wait()
    oh[1].wait()


def kernel(input, weight):
    mesh = plsc.VectorSubcoreMesh(core_axis_name="c", subcore_axis_name="s")
    out_t = pl.kernel(
        _emb_body,
        out_type=jax.ShapeDtypeStruct((HIST, DIM, BATCH), jnp.float32),
        mesh=mesh,
        scratch_types=[
            pltpu.VMEM((VOCAB,), jnp.float32),
            pltpu.VMEM((BATCH,), jnp.int32),
            pltpu.VMEM((BATCH,), jnp.int32),
            pltpu.VMEM((BATCH,), jnp.float32),
            pltpu.VMEM((BATCH,), jnp.float32),
            pltpu.SemaphoreType.DMA,
            pltpu.SemaphoreType.DMA,
            pltpu.SemaphoreType.DMA,
            pltpu.SemaphoreType.DMA,
        ],
        compiler_params=pltpu.CompilerParams(use_tc_tiling_on_sc=False,
                                             needs_layout_passes=False),
    )(input.T, weight.T)
    return out_t.transpose(2, 0, 1)


# speedup vs baseline: 1.5334x; 1.2927x over previous
import jax
import jax.numpy as jnp
from jax import lax
from jax.experimental import pallas as pl
from jax.experimental.pallas import tpu as pltpu
from jax.experimental.pallas import tpu_sc as plsc

BATCH = 4096
HIST = 50
DIM = 64
VOCAB = 100000
NUM_CORES = 2
NUM_SUBCORES = 16
NW = NUM_CORES * NUM_SUBCORES
FEAT_W = DIM // NW
NCHUNK = BATCH // 16
UNROLL = 8


def _emb_body(idx_hbm, table_hbm, out_hbm, ish, wrow,
              ib0, ib1, ob0, ob1, is0, is1, os0, os1):
    cid = lax.axis_index("c")
    sid = lax.axis_index("s")
    w = sid * NUM_CORES + cid

    @pl.when(sid == 0)
    def _():
        pltpu.sync_copy(idx_hbm, ish)
    plsc.subcore_barrier()

    ibufs = (ib0, ib1)
    obufs = (ob0, ob1)
    isems = (is0, is1)
    osems = (os0, os1)
    ih = [None, None]
    oh = [None, None]

    for dd in range(FEAT_W):
        d = FEAT_W * w + dd
        pltpu.sync_copy(table_hbm.at[d], wrow)
        ih[0] = pltpu.async_copy(ish.at[0], ibufs[0], isems[0])
        for h in range(HIST):
            par = h % 2
            if h + 1 < HIST:
                ih[1 - par] = pltpu.async_copy(
                    ish.at[h + 1], ibufs[1 - par], isems[1 - par])
            ih[par].wait()
            if oh[par] is not None:
                oh[par].wait()
            idx_v = ibufs[par]
            out_v = obufs[par]

            @plsc.parallel_loop(0, BATCH, step=16, unroll=UNROLL)
            def _gather(off, idx_v=idx_v, out_v=out_v):
                i16 = idx_v[pl.ds(off, 16)]
                out_v[pl.ds(off, 16)] = plsc.load_gather(wrow, [i16])
            oh[par] = pltpu.async_copy(out_v, out_hbm.at[h, d], osems[par])
    oh[0].wait()
    oh[1].wait()


def kernel(input, weight):
    mesh = plsc.VectorSubcoreMesh(core_axis_name="c", subcore_axis_name="s")
    out_t = pl.kernel(
        _emb_body,
        out_type=jax.ShapeDtypeStruct((HIST, DIM, BATCH), jnp.float32),
        mesh=mesh,
        scratch_types=[
            pltpu.VMEM_SHARED((HIST, BATCH), jnp.int32),
            pltpu.VMEM((VOCAB,), jnp.float32),
            pltpu.VMEM((BATCH,), jnp.int32),
            pltpu.VMEM((BATCH,), jnp.int32),
            pltpu.VMEM((BATCH,), jnp.float32),
            pltpu.VMEM((BATCH,), jnp.float32),
            pltpu.SemaphoreType.DMA,
            pltpu.SemaphoreType.DMA,
            pltpu.SemaphoreType.DMA,
            pltpu.SemaphoreType.DMA,
        ],
        compiler_params=pltpu.CompilerParams(use_tc_tiling_on_sc=False,
                                             needs_layout_passes=False),
    )(input.T, weight.T)
    return out_t.transpose(2, 0, 1)


# speedup vs baseline: 2.2760x; 1.4842x over previous
import jax
import jax.numpy as jnp
from jax import lax
from jax.experimental import pallas as pl
from jax.experimental.pallas import tpu as pltpu
from jax.experimental.pallas import tpu_sc as plsc

BATCH = 4096
HIST = 50
DIM = 64
VOCAB = 100000
NUM_CORES = 2
NUM_SUBCORES = 16
NW = NUM_CORES * NUM_SUBCORES
FEAT_W = DIM // NW
NCHUNK = BATCH // 16
UNROLL = 8


def _emb_body(idx_hbm, table_hbm, out_hbm, ish, wrow,
              ib0, ib1, ob0, ob1, is0, is1, os0, os1):
    cid = lax.axis_index("c")
    sid = lax.axis_index("s")
    w = sid * NUM_CORES + cid

    @pl.when(sid == 0)
    def _():
        pltpu.sync_copy(idx_hbm, ish)
    plsc.subcore_barrier()

    ibufs = (ib0, ib1)
    obufs = (ob0, ob1)
    isems = (is0, is1)
    osems = (os0, os1)

    def wait_idx(p):
        pltpu.make_async_copy(ish.at[0], ibufs[p], isems[p]).wait()

    def wait_out(p, dst):
        pltpu.make_async_copy(obufs[p], dst, osems[p]).wait()

    def do_h(h, dg, dr, p):
        wait_idx(p)
        idx_v = ibufs[p]
        out_v = obufs[p]

        @plsc.parallel_loop(0, BATCH // 128, step=1, unroll=2)
        def _gather(t, idx_v=idx_v, out_v=out_v):
            for c in range(8):
                i16 = idx_v[pl.ds(t * 128 + c * 16, 16)]
                out_v[t, pl.ds(c * 16, 16)] = plsc.load_gather(wrow, [i16])

        pltpu.async_copy(out_v, out_hbm.at[h, dg, :, dr], osems[p])

    for dd in range(FEAT_W):
        d = FEAT_W * w + dd
        dg = d // 8
        dr = d % 8
        pltpu.sync_copy(table_hbm.at[d], wrow)
        pltpu.async_copy(ish.at[0], ibufs[0], isems[0])
        pltpu.async_copy(ish.at[1], ibufs[1], isems[1])

        def pair(j, _, dg=dg, dr=dr):
            h0 = 2 * j

            @pl.when(j > 0)
            def _():
                wait_out(0, out_hbm.at[h0, dg, :, dr])
            do_h(h0, dg, dr, 0)

            @pl.when(j < HIST // 2 - 1)
            def _():
                pltpu.async_copy(ish.at[h0 + 2], ibufs[0], isems[0])

            @pl.when(j > 0)
            def _():
                wait_out(1, out_hbm.at[h0, dg, :, dr])
            do_h(h0 + 1, dg, dr, 1)

            @pl.when(j < HIST // 2 - 1)
            def _():
                pltpu.async_copy(ish.at[h0 + 3], ibufs[1], isems[1])
            return ()

        lax.fori_loop(0, HIST // 2, pair, ())
        wait_out(0, out_hbm.at[0, dg, :, dr])
        wait_out(1, out_hbm.at[0, dg, :, dr])


def kernel(input, weight):
    mesh = plsc.VectorSubcoreMesh(core_axis_name="c", subcore_axis_name="s")
    out5 = pl.kernel(
        _emb_body,
        out_type=jax.ShapeDtypeStruct((HIST, 8, BATCH // 128, 8, 128),
                                      jnp.float32),
        mesh=mesh,
        scratch_types=[
            pltpu.VMEM_SHARED((HIST, BATCH), jnp.int32),
            pltpu.VMEM((VOCAB,), jnp.float32),
            pltpu.VMEM((BATCH,), jnp.int32),
            pltpu.VMEM((BATCH,), jnp.int32),
            pltpu.VMEM((BATCH // 128, 128), jnp.float32),
            pltpu.VMEM((BATCH // 128, 128), jnp.float32),
            pltpu.SemaphoreType.DMA,
            pltpu.SemaphoreType.DMA,
            pltpu.SemaphoreType.DMA,
            pltpu.SemaphoreType.DMA,
        ],
        compiler_params=pltpu.CompilerParams(use_tc_tiling_on_sc=False,
                                             needs_layout_passes=False),
    )(input.T, weight.T)
    return out5.transpose(2, 4, 0, 1, 3).reshape(BATCH, HIST, DIM)


# speedup vs baseline: 2.2805x; 1.0020x over previous
import jax
import jax.numpy as jnp
from jax import lax
from jax.experimental import pallas as pl
from jax.experimental.pallas import tpu as pltpu
from jax.experimental.pallas import tpu_sc as plsc

BATCH = 4096
HIST = 50
DIM = 64
VOCAB = 100000
NUM_CORES = 2
NUM_SUBCORES = 16
NW = NUM_CORES * NUM_SUBCORES
FEAT_W = DIM // NW


def _emb_body(idx_hbm, table_hbm, out_hbm, ish, wrow,
              ib0, ib1, ob0, ob1, is0, is1, os0, os1):
    cid = lax.axis_index("c")
    sid = lax.axis_index("s")
    w = sid * NUM_CORES + cid

    @pl.when(sid == 0)
    def _():
        pltpu.sync_copy(idx_hbm, ish)
    plsc.subcore_barrier()

    ibufs = (ib0, ib1)
    obufs = (ob0, ob1)
    isems = (is0, is1)
    osems = (os0, os1)

    def wait_idx(p):
        pltpu.make_async_copy(ish.at[0], ibufs[p], isems[p]).wait()

    def wait_out(p, dst):
        pltpu.make_async_copy(obufs[p], dst, osems[p]).wait()

    def do_h(h, dg, dr, p):
        wait_idx(p)
        idx_v = ibufs[p]
        out_v = obufs[p]

        @plsc.parallel_loop(0, BATCH // 128, step=1, unroll=2)
        def _gather(t, idx_v=idx_v, out_v=out_v):
            for c in range(8):
                i16 = idx_v[pl.ds(t * 128 + c * 16, 16)]
                out_v[t, pl.ds(c * 16, 16)] = plsc.load_gather(wrow, [i16])

        pltpu.async_copy(out_v, out_hbm.at[h, dg, :, dr], osems[p])

    for dd in range(FEAT_W):
        d = FEAT_W * w + dd
        dg = d // 8
        dr = d % 8
        pltpu.sync_copy(table_hbm.at[d], wrow)
        pltpu.async_copy(ish.at[0], ibufs[0], isems[0])
        pltpu.async_copy(ish.at[1], ibufs[1], isems[1])

        def pair(j, _, dg=dg, dr=dr):
            h0 = 2 * j

            @pl.when(j > 0)
            def _():
                wait_out(0, out_hbm.at[h0, dg, :, dr])
            do_h(h0, dg, dr, 0)

            @pl.when(j < HIST // 2 - 1)
            def _():
                pltpu.async_copy(ish.at[h0 + 2], ibufs[0], isems[0])

            @pl.when(j > 0)
            def _():
                wait_out(1, out_hbm.at[h0, dg, :, dr])
            do_h(h0 + 1, dg, dr, 1)

            @pl.when(j < HIST // 2 - 1)
            def _():
                pltpu.async_copy(ish.at[h0 + 3], ibufs[1], isems[1])
            return ()

        lax.fori_loop(0, HIST // 2, pair, ())
        wait_out(0, out_hbm.at[0, dg, :, dr])
        wait_out(1, out_hbm.at[0, dg, :, dr])


def kernel(input, weight):
    mesh = plsc.VectorSubcoreMesh(core_axis_name="c", subcore_axis_name="s")
    out5 = pl.kernel(
        _emb_body,
        out_type=jax.ShapeDtypeStruct((HIST, 8, BATCH // 128, 8, 128),
                                      jnp.float32),
        mesh=mesh,
        scratch_types=[
            pltpu.VMEM_SHARED((HIST, BATCH), jnp.int32),
            pltpu.VMEM((VOCAB,), jnp.float32),
            pltpu.VMEM((BATCH,), jnp.int32),
            pltpu.VMEM((BATCH,), jnp.int32),
            pltpu.VMEM((BATCH // 128, 128), jnp.float32),
            pltpu.VMEM((BATCH // 128, 128), jnp.float32),
            pltpu.SemaphoreType.DMA,
            pltpu.SemaphoreType.DMA,
            pltpu.SemaphoreType.DMA,
            pltpu.SemaphoreType.DMA,
        ],
        compiler_params=pltpu.CompilerParams(use_tc_tiling_on_sc=False,
                                             needs_layout_passes=False),
    )(input.T, weight.T)
    return out5.transpose(2, 4, 0, 1, 3).reshape(BATCH, HIST, DIM)
